# baseline (speedup 1.0000x reference)
import functools

import jax
import jax.numpy as jnp
from jax import lax
from jax.experimental import pallas as pl
from jax.experimental.pallas import tpu as pltpu
from jax.experimental.pallas import tpu_sc as plsc

NC, NS, L = 2, 16, 16
NW = NC * NS
B, C, H, W = 4, 96, 384, 384
SP = H * W
CHUNK = 512
G = CHUNK // L
PER_W = SP // NW
NCHUNK = PER_W // CHUNK
GPW = PER_W // L
SPG = SP // L

_mesh = plsc.VectorSubcoreMesh(
    core_axis_name="core", subcore_axis_name="subcore",
    num_cores=NC, num_subcores=NS)


def _worker_id():
  return lax.axis_index("subcore") * NC + lax.axis_index("core")


def _stats_body(x_hbm, s1_hbm, s2_hbm, cnt_hbm, mask_hbm,
                buf, mbuf, acc1, acc2, cntbuf):
  wid = _worker_id()
  zero = jnp.zeros((L,), jnp.float32)
  one = jnp.ones((L,), jnp.float32)

  def zacc(c, carry):
    acc1[c, :] = zero
    acc2[c, :] = zero
    return carry
  lax.fori_loop(0, C, zacc, 0)

  cnt_total = zero
  for b in range(B):
    for j in range(NCHUNK):
      start = pl.multiple_of(wid * PER_W + j * CHUNK, CHUNK)
      gstart = pl.multiple_of(wid * GPW + j * G, G)
      pltpu.sync_copy(x_hbm.at[b, :, pl.ds(start, CHUNK)], buf)

      def g_body(g, cnt):
        off = pl.multiple_of(g * L, L)
        def c_body(c, m):
          v = buf[c, pl.ds(off, L)]
          return jnp.where(v > 0.0, one, m)
        m = lax.fori_loop(0, C, c_body, zero)
        mbuf[g, :] = m
        return cnt + m
      cnt_total = lax.fori_loop(0, G, g_body, cnt_total)
      pltpu.sync_copy(mbuf, mask_hbm.at[b, pl.ds(gstart, G)])

      def c_body2(c, carry):
        def g_body2(g, sums):
          s1, s2 = sums
          off = pl.multiple_of(g * L, L)
          v = buf[c, pl.ds(off, L)]
          m = mbuf[g, :]
          vm = v * m
          return (s1 + vm, s2 + vm * v)
        s1, s2 = lax.fori_loop(0, G, g_body2, (zero, zero))
        acc1[c, :] = acc1[c, :] + s1
        acc2[c, :] = acc2[c, :] + s2
        return carry
      lax.fori_loop(0, C, c_body2, 0)

  cntbuf[:] = cnt_total
  pltpu.sync_copy(acc1, s1_hbm.at[wid])
  pltpu.sync_copy(acc2, s2_hbm.at[wid])
  pltpu.sync_copy(cntbuf, cnt_hbm.at[wid])


_stats_call = functools.partial(
    pl.kernel,
    out_type=[
        jax.ShapeDtypeStruct((NW, C, L), jnp.float32),
        jax.ShapeDtypeStruct((NW, C, L), jnp.float32),
        jax.ShapeDtypeStruct((NW, L), jnp.float32),
        jax.ShapeDtypeStruct((B, SPG, L), jnp.float32),
    ],
    mesh=_mesh,
    scratch_types=[
        pltpu.VMEM((C, CHUNK), jnp.float32),
        pltpu.VMEM((G, L), jnp.float32),
        pltpu.VMEM((C, L), jnp.float32),
        pltpu.VMEM((C, L), jnp.float32),
        pltpu.VMEM((L,), jnp.float32),
    ],
)(_stats_body)


def _norm_body(x_hbm, mask_hbm, scale_hbm, shift_hbm, out_hbm,
               buf, mbuf, scb, shb):
  wid = _worker_id()
  pltpu.sync_copy(scale_hbm, scb)
  pltpu.sync_copy(shift_hbm, shb)
  for b in range(B):
    for j in range(NCHUNK):
      start = pl.multiple_of(wid * PER_W + j * CHUNK, CHUNK)
      gstart = pl.multiple_of(wid * GPW + j * G, G)
      pltpu.sync_copy(x_hbm.at[b, :, pl.ds(start, CHUNK)], buf)
      pltpu.sync_copy(mask_hbm.at[b, pl.ds(gstart, G)], mbuf)

      def c_body(c, carry):
        sc = scb[c, :]
        sh = shb[c, :]
        def g_body(g, inner):
          off = pl.multiple_of(g * L, L)
          v = buf[c, pl.ds(off, L)]
          m = mbuf[g, :]
          y = jnp.where(m > 0.0, v * sc + sh, v)
          buf[c, pl.ds(off, L)] = y
          return inner
        lax.fori_loop(0, G, g_body, 0)
        return carry
      lax.fori_loop(0, C, c_body, 0)

      pltpu.sync_copy(buf, out_hbm.at[b, :, pl.ds(start, CHUNK)])


_norm_call = functools.partial(
    pl.kernel,
    out_type=jax.ShapeDtypeStruct((B, C, SP), jnp.float32),
    mesh=_mesh,
    scratch_types=[
        pltpu.VMEM((C, CHUNK), jnp.float32),
        pltpu.VMEM((G, L), jnp.float32),
        pltpu.VMEM((C, L), jnp.float32),
        pltpu.VMEM((C, L), jnp.float32),
    ],
)(_norm_body)


def kernel(input, gamma, beta):
  x3 = input.reshape(B, C, SP)
  s1p, s2p, cntp, maskp = _stats_call(x3)
  count = jnp.maximum(jnp.sum(cntp), 1.0)
  mean = jnp.sum(s1p, axis=(0, 2)) / count
  var = jnp.maximum(jnp.sum(s2p, axis=(0, 2)) / count - mean * mean, 0.0)
  scale = gamma * lax.rsqrt(var + 1e-5)
  shift = beta - mean * scale
  scale_b = jnp.tile(scale[:, None], (1, L))
  shift_b = jnp.tile(shift[:, None], (1, L))
  out = _norm_call(x3, maskp, scale_b, shift_b)
  return out.reshape(B, C, H, W)
